# baseline (speedup 1.0000x reference)
import jax
import jax.numpy as jnp
from jax.experimental import pallas as pl
from jax.experimental.pallas import tpu as pltpu

HID = 4096
NE = 64
BT = 512


def _router_body(x_ref, w_ref, b_ref, o_ref):
    x = x_ref[...].astype(jnp.bfloat16)
    w = w_ref[...].astype(jnp.bfloat16)
    logits = jax.lax.dot_general(
        x, w, (((1,), (1,)), ((), ())), preferred_element_type=jnp.float32
    )
    logits = logits + b_ref[...]
    m = jnp.max(logits, axis=-1, keepdims=True)
    e = jnp.exp(logits - m)
    o_ref[...] = e / jnp.sum(e, axis=-1, keepdims=True)


def kernel(x, W, b):
    tokens = x.shape[0]
    return pl.pallas_call(
        _router_body,
        grid=(tokens // BT,),
        in_specs=[
            pl.BlockSpec((BT, HID), lambda i: (i, 0)),
            pl.BlockSpec((NE, HID), lambda i: (0, 0)),
            pl.BlockSpec((1, NE), lambda i: (0, 0)),
        ],
        out_specs=pl.BlockSpec((BT, NE), lambda i: (i, 0)),
        out_shape=jax.ShapeDtypeStruct((tokens, NE), jnp.float32),
        compiler_params=pltpu.CompilerParams(
            dimension_semantics=("parallel",),
        ),
    )(x, W, b.reshape(1, NE))


# speedup vs baseline: 1.0157x; 1.0157x over previous
import jax
import jax.numpy as jnp
from jax.experimental import pallas as pl
from jax.experimental.pallas import tpu as pltpu

HID = 4096
NE = 64
BT = 1024


def _router_body(x_ref, w_ref, b_ref, o_ref):
    x = x_ref[...]
    w = w_ref[...]
    logits = jax.lax.dot_general(
        x, w, (((1,), (1,)), ((), ())), preferred_element_type=jnp.float32
    )
    logits = logits + b_ref[...]
    m = jnp.max(logits, axis=-1, keepdims=True)
    e = jnp.exp(logits - m)
    o_ref[...] = e / jnp.sum(e, axis=-1, keepdims=True)


def kernel(x, W, b):
    tokens = x.shape[0]
    return pl.pallas_call(
        _router_body,
        grid=(tokens // BT,),
        in_specs=[
            pl.BlockSpec((BT, HID), lambda i: (i, 0)),
            pl.BlockSpec((NE, HID), lambda i: (0, 0)),
            pl.BlockSpec((1, NE), lambda i: (0, 0)),
        ],
        out_specs=pl.BlockSpec((BT, NE), lambda i: (i, 0)),
        out_shape=jax.ShapeDtypeStruct((tokens, NE), jnp.float32),
        compiler_params=pltpu.CompilerParams(
            dimension_semantics=("parallel",),
        ),
    )(x, W, b.reshape(1, NE))


# speedup vs baseline: 1.0467x; 1.0305x over previous
import jax
import jax.numpy as jnp
from jax.experimental import pallas as pl
from jax.experimental.pallas import tpu as pltpu

HID = 4096
NE = 64
CH = 512
NBUF = 4


def _router_body(x_hbm, w_ref, b_ref, o_hbm, xbuf, obuf, insem, outsem):
    w = w_ref[...]
    bb = b_ref[...]
    nch = x_hbm.shape[0] // CH

    for s in range(NBUF):
        pltpu.make_async_copy(
            x_hbm.at[pl.ds(s * CH, CH)], xbuf.at[s], insem.at[s]
        ).start()

    def outer(g, _):
        base = g * NBUF
        for s in range(NBUF):
            i = base + s
            pltpu.make_async_copy(
                x_hbm.at[pl.ds(i * CH, CH)], xbuf.at[s], insem.at[s]
            ).wait()
            x = xbuf[s]
            logits = jax.lax.dot_general(
                x, w, (((1,), (1,)), ((), ())),
                preferred_element_type=jnp.float32,
            ) + bb
            m = jnp.max(logits, axis=-1, keepdims=True)
            e = jnp.exp(logits - m)
            res = e / jnp.sum(e, axis=-1, keepdims=True)

            @pl.when(g > 0)
            def _():
                pltpu.make_async_copy(
                    obuf.at[s], o_hbm.at[pl.ds((i - NBUF) * CH, CH)], outsem.at[s]
                ).wait()

            obuf[s] = res
            pltpu.make_async_copy(
                obuf.at[s], o_hbm.at[pl.ds(i * CH, CH)], outsem.at[s]
            ).start()

            @pl.when(i + NBUF < nch)
            def _():
                pltpu.make_async_copy(
                    x_hbm.at[pl.ds((i + NBUF) * CH, CH)], xbuf.at[s], insem.at[s]
                ).start()

        return _

    jax.lax.fori_loop(0, nch // NBUF, outer, None)

    for s in range(NBUF):
        pltpu.make_async_copy(
            obuf.at[s], o_hbm.at[pl.ds((nch - NBUF + s) * CH, CH)], outsem.at[s]
        ).wait()


def kernel(x, W, b):
    tokens = x.shape[0]
    return pl.pallas_call(
        _router_body,
        in_specs=[
            pl.BlockSpec(memory_space=pl.ANY),
            pl.BlockSpec((NE, HID), lambda: (0, 0)),
            pl.BlockSpec((1, NE), lambda: (0, 0)),
        ],
        out_specs=pl.BlockSpec(memory_space=pl.ANY),
        out_shape=jax.ShapeDtypeStruct((tokens, NE), jnp.float32),
        scratch_shapes=[
            pltpu.VMEM((NBUF, CH, HID), jnp.float32),
            pltpu.VMEM((NBUF, CH, NE), jnp.float32),
            pltpu.SemaphoreType.DMA((NBUF,)),
            pltpu.SemaphoreType.DMA((NBUF,)),
        ],
    )(x, W, b.reshape(1, NE))
